# baseline (device time: 14117 ns/iter reference)
import jax
import jax.numpy as jnp
from jax import lax
from jax.experimental import pallas as pl
from jax.experimental.pallas import tpu as pltpu

N_CHUNKS = 8


def kernel(x):
    m, n = x.shape
    half = n // 2
    hr = m // 2
    ch = hr // N_CHUNKS

    def body(x_ref, out_ref, send_buf,
             x_send_sems, x_recv_sems, z_send_sems, z_recv_sems):
        my_x = lax.axis_index("x")
        my_y = lax.axis_index("y")
        my_z = lax.axis_index("z")
        other = 1 - my_x
        q = my_z % 2
        zb = my_z ^ 1

        send_buf[...] = x_ref[
            pl.ds(q * hr, hr), pl.ds(other * half, half)
        ].astype(jnp.bfloat16)

        barrier_sem = pltpu.get_barrier_semaphore()
        for tgt in ((other, my_y, my_z), (my_x, my_y, zb)):
            pl.semaphore_signal(
                barrier_sem, inc=1,
                device_id=tgt, device_id_type=pl.DeviceIdType.MESH,
            )
        pl.semaphore_wait(barrier_sem, 2)

        x_rdmas = []
        for c in range(N_CHUNKS):
            r = pltpu.make_async_remote_copy(
                src_ref=send_buf.at[pl.ds(c * ch, ch), :],
                dst_ref=out_ref.at[pl.ds(my_x * m + q * hr + c * ch, ch), :],
                send_sem=x_send_sems.at[c],
                recv_sem=x_recv_sems.at[c],
                device_id=(other, my_y, my_z),
                device_id_type=pl.DeviceIdType.MESH,
            )
            r.start()
            x_rdmas.append(r)

        out_ref[pl.ds(my_x * m, m), :] = x_ref[:, pl.ds(my_x * half, half)].astype(
            jnp.bfloat16
        )

        out_ref[pl.ds(other * m + (1 - q) * hr, hr), :] = send_buf[...]

        for c in range(N_CHUNKS):
            x_rdmas[c].wait_recv()
        for c in range(N_CHUNKS):
            x_rdmas[c].wait_send()

    return pl.pallas_call(
        body,
        out_shape=jax.ShapeDtypeStruct((2 * m, half), jnp.bfloat16),
        in_specs=[pl.BlockSpec(memory_space=pltpu.VMEM)],
        out_specs=pl.BlockSpec(memory_space=pltpu.VMEM),
        scratch_shapes=[
            pltpu.VMEM((hr, half), jnp.bfloat16),
            pltpu.SemaphoreType.DMA((N_CHUNKS,)),
            pltpu.SemaphoreType.DMA((N_CHUNKS,)),
            pltpu.SemaphoreType.DMA((N_CHUNKS,)),
            pltpu.SemaphoreType.DMA((N_CHUNKS,)),
        ],
        compiler_params=pltpu.CompilerParams(collective_id=0),
    )(x)
